# Initial kernel scaffold; baseline (speedup 1.0000x reference)
import jax
import jax.numpy as jnp
from jax import lax
from jax.experimental import pallas as pl
from jax.experimental.pallas import tpu as pltpu
from jax.experimental.pallas import tpu_sc as plsc

N = 10000
E = 320000
D = 128
C = 4
OUT = D * C

NC = 2
NS = 16
L = 16

RANGE = 2560
GARB = 128
ACC_ROWS = RANGE + GARB
ROWS_PER_TILE = ACC_ROWS // NS
EDGES_PER_TILE = E // NS
BLK = 80
NBLK = EDGES_PER_TILE // BLK


def _mm_body(x_ref, w_ref, o_ref):
    o_ref[...] = jnp.dot(x_ref[...], w_ref[...],
                         preferred_element_type=jnp.float32)


def _project(x, w):
    return pl.pallas_call(
        _mm_body,
        grid=(10,),
        in_specs=[
            pl.BlockSpec((N // 10, D), lambda i: (i, 0)),
            pl.BlockSpec((D, D), lambda i: (0, 0)),
        ],
        out_specs=pl.BlockSpec((N // 10, D), lambda i: (i, 0)),
        out_shape=jax.ShapeDtypeStruct((N, D), jnp.float32),
    )(x, w)


def _sc_kernel(h, src, dst, attr_flat, b):
    mesh = plsc.VectorSubcoreMesh(core_axis_name="c", subcore_axis_name="s",
                                  num_cores=NC, num_subcores=NS)

    def body(h_hbm, src_hbm, dst_hbm, attr_hbm, b_hbm, out_hbm,
             acc, srcb, dstb, offb, attrb, rows, msg, chunk, biasv, dsem):
        core = lax.axis_index("c")
        s = lax.axis_index("s")
        pltpu.sync_copy(b_hbm, biasv)
        zeros16 = jnp.zeros((L,), jnp.float32)

        for p in range(2):
            nb = (2 * core + p) * RANGE

            def zrow(r, _):
                for j in range(OUT // L):
                    chunk[r, pl.ds(L * j, L)] = zeros16
                return 0
            lax.fori_loop(0, 16, zrow, 0)

            def zcp(k, carry):
                pltpu.sync_copy(chunk,
                                acc.at[pl.ds(ROWS_PER_TILE * s + 16 * k, 16), :])
                return carry
            lax.fori_loop(0, ROWS_PER_TILE // 16, zcp, 0)
            pltpu.sync_copy(
                chunk.at[pl.ds(0, ROWS_PER_TILE % 16), :],
                acc.at[pl.ds(ROWS_PER_TILE * s + 16 * (ROWS_PER_TILE // 16),
                             ROWS_PER_TILE % 16), :])
            plsc.subcore_barrier()

            nb_vec = jnp.full((L,), nb, dtype=jnp.int32)

            def edge_body(e, carry):
                ev = jnp.full((L,), e, dtype=jnp.int32)
                row = [rows[e, pl.ds(L * j, L)] for j in range(D // L)]
                for c in range(C):
                    scale = plsc.load_gather(attrb, [ev * C + c])
                    for j in range(D // L):
                        msg[e, pl.ds(c * D + L * j, L)] = row[j] * scale
                return carry

            def blk_body(blk, carry):
                base = s * EDGES_PER_TILE + blk * BLK
                pltpu.sync_copy(src_hbm.at[pl.ds(base, BLK)], srcb)
                pltpu.sync_copy(dst_hbm.at[pl.ds(base, BLK)], dstb)
                pltpu.sync_copy(attr_hbm.at[pl.ds(base * C, BLK * C)], attrb)
                for g in range(BLK // L):
                    off = dstb[pl.ds(L * g, L)] - nb_vec
                    m = (off >= 0) & (off < RANGE)
                    garb = RANGE + (off & (GARB - 1))
                    offb[pl.ds(L * g, L)] = jnp.where(m, off, garb)
                pltpu.async_copy(h_hbm.at[srcb], rows, dsem).wait()
                lax.fori_loop(0, BLK, edge_body, 0)
                pltpu.sync_copy(msg, acc.at[offb], add=True)
                return carry

            lax.fori_loop(0, NBLK, blk_body, 0)
            plsc.subcore_barrier()

            limit = jnp.minimum(RANGE, N - nb)
            valid = jnp.clip(limit - ROWS_PER_TILE * s, 0, ROWS_PER_TILE)
            nfull = valid // 16
            rem = valid - 16 * nfull

            def bias_relu_rows(nrows_ref_unused):
                pass

            def chunk_body(k, carry):
                r0 = ROWS_PER_TILE * s + 16 * k
                pltpu.sync_copy(acc.at[pl.ds(r0, 16), :], chunk)

                def row_body(r, c2):
                    for j in range(OUT // L):
                        v = chunk[r, pl.ds(L * j, L)] + biasv[pl.ds(L * j, L)]
                        chunk[r, pl.ds(L * j, L)] = jnp.maximum(v, 0.0)
                    return c2
                lax.fori_loop(0, 16, row_body, 0)
                pltpu.sync_copy(chunk, out_hbm.at[pl.ds(nb + r0, 16), :])
                return carry
            lax.fori_loop(0, nfull, chunk_body, 0)

            def rem_body(r, carry):
                r0 = ROWS_PER_TILE * s + 16 * nfull + r
                pltpu.sync_copy(acc.at[pl.ds(r0, 1), :], chunk.at[pl.ds(0, 1), :])
                for j in range(OUT // L):
                    v = chunk[0, pl.ds(L * j, L)] + biasv[pl.ds(L * j, L)]
                    chunk[0, pl.ds(L * j, L)] = jnp.maximum(v, 0.0)
                pltpu.sync_copy(chunk.at[pl.ds(0, 1), :],
                                out_hbm.at[pl.ds(nb + r0, 1), :])
                return carry
            lax.fori_loop(0, rem, rem_body, 0)

    f = pl.kernel(
        body,
        out_type=jax.ShapeDtypeStruct((N, OUT), jnp.float32),
        mesh=mesh,
        scratch_types=dict(
            acc=pltpu.VMEM_SHARED((ACC_ROWS, OUT), jnp.float32),
            srcb=pltpu.VMEM((BLK,), jnp.int32),
            dstb=pltpu.VMEM((BLK,), jnp.int32),
            offb=pltpu.VMEM((BLK,), jnp.int32),
            attrb=pltpu.VMEM((BLK * C,), jnp.float32),
            rows=pltpu.VMEM((BLK, D), jnp.float32),
            msg=pltpu.VMEM((BLK, OUT), jnp.float32),
            chunk=pltpu.VMEM((16, OUT), jnp.float32),
            biasv=pltpu.VMEM((OUT,), jnp.float32),
            dsem=pltpu.SemaphoreType.DMA,
        ),
    )
    return f(h, src, dst, attr_flat, b)


def kernel(node_features, edge_index, edge_attr, W, b):
    h = _project(node_features, W)
    src = edge_index[0].astype(jnp.int32)
    dst = edge_index[1].astype(jnp.int32)
    attr_flat = edge_attr.reshape(-1)
    return _sc_kernel(h, src, dst, attr_flat, b)


# speedup vs baseline: 1.3542x; 1.3542x over previous
import jax
import jax.numpy as jnp
from jax import lax
from jax.experimental import pallas as pl
from jax.experimental.pallas import tpu as pltpu
from jax.experimental.pallas import tpu_sc as plsc

N = 10000
E = 320000
D = 128
C = 4
OUT = D * C

NC = 2
NS = 16
L = 16

EDGES_PER_TILE = E // NS
BLK = 80
NBLK = EDGES_PER_TILE // BLK
NZCH = N // BLK
PASSES = C // NC


def _mm_body(x_ref, w_ref, o_ref):
    o_ref[...] = jnp.dot(x_ref[...], w_ref[...],
                         preferred_element_type=jnp.float32)


def _project(x, w):
    return pl.pallas_call(
        _mm_body,
        grid=(10,),
        in_specs=[
            pl.BlockSpec((N // 10, D), lambda i: (i, 0)),
            pl.BlockSpec((D, D), lambda i: (0, 0)),
        ],
        out_specs=pl.BlockSpec((N // 10, D), lambda i: (i, 0)),
        out_shape=jax.ShapeDtypeStruct((N, D), jnp.float32),
    )(x, w)


def _sc_scatter(h, src, dst, attr_flat):
    mesh = plsc.VectorSubcoreMesh(core_axis_name="c", subcore_axis_name="s",
                                  num_cores=NC, num_subcores=NS)

    def body(h_hbm, src_hbm, dst_hbm, attr_hbm, out_hbm,
             acc, srcb, dstb, attrb, rows, msg, zbuf, dsem):
        core = lax.axis_index("c")
        s = lax.axis_index("s")
        zeros16 = jnp.zeros((L,), jnp.float32)

        def zrow(r, _):
            for j in range(D // L):
                zbuf[r, pl.ds(L * j, L)] = zeros16
            return 0
        lax.fori_loop(0, BLK, zrow, 0)

        for p in range(PASSES):
            ch = PASSES * core + p
            chv = jnp.full((L,), 1, dtype=jnp.int32) * ch

            def zcp(i, carry):
                k = s + NS * i

                @pl.when(k < NZCH)
                def _():
                    off = pl.multiple_of(k * BLK, 8)
                    pltpu.sync_copy(zbuf, acc.at[pl.ds(off, BLK), :])
                return carry
            lax.fori_loop(0, (NZCH + NS - 1) // NS, zcp, 0)
            plsc.subcore_barrier()

            def edge_body(e, carry):
                ev = jnp.full((L,), e, dtype=jnp.int32)
                scale = plsc.load_gather(attrb, [ev * C + chv])
                for j in range(D // L):
                    msg[e, pl.ds(L * j, L)] = rows[e, pl.ds(L * j, L)] * scale
                return carry

            def blk_body(blk, carry):
                base = s * EDGES_PER_TILE + blk * BLK
                pltpu.sync_copy(src_hbm.at[pl.ds(base, BLK)], srcb)
                pltpu.sync_copy(dst_hbm.at[pl.ds(base, BLK)], dstb)
                pltpu.sync_copy(attr_hbm.at[pl.ds(base * C, BLK * C)], attrb)
                pltpu.async_copy(h_hbm.at[srcb], rows, dsem).wait()
                lax.fori_loop(0, BLK, edge_body, 0)
                pltpu.sync_copy(msg, acc.at[dstb], add=True)
                return carry

            lax.fori_loop(0, NBLK, blk_body, 0)
            plsc.subcore_barrier()

            def wcp(i, carry):
                k = s + NS * i

                @pl.when(k < NZCH)
                def _():
                    off = pl.multiple_of(k * BLK, 8)
                    dof = pl.multiple_of(ch * N + k * BLK, 8)
                    pltpu.sync_copy(acc.at[pl.ds(off, BLK), :],
                                    out_hbm.at[pl.ds(dof, BLK), :])
                return carry
            lax.fori_loop(0, (NZCH + NS - 1) // NS, wcp, 0)
            if p + 1 < PASSES:
                plsc.subcore_barrier()

    f = pl.kernel(
        body,
        out_type=jax.ShapeDtypeStruct((C * N, D), jnp.float32),
        mesh=mesh,
        compiler_params=pltpu.CompilerParams(needs_layout_passes=False),
        scratch_types=dict(
            acc=pltpu.VMEM_SHARED((N, D), jnp.float32),
            srcb=pltpu.VMEM((BLK,), jnp.int32),
            dstb=pltpu.VMEM((BLK,), jnp.int32),
            attrb=pltpu.VMEM((BLK * C,), jnp.float32),
            rows=pltpu.VMEM((BLK, D), jnp.float32),
            msg=pltpu.VMEM((BLK, D), jnp.float32),
            zbuf=pltpu.VMEM((BLK, D), jnp.float32),
            dsem=pltpu.SemaphoreType.DMA,
        ),
    )
    return f(h, src, dst, attr_flat)


def _fin_body(a0, a1, a2, a3, b_ref, o_ref):
    av = [a0, a1, a2, a3]
    for c in range(C):
        o_ref[:, c * D:(c + 1) * D] = jnp.maximum(
            av[c][...] + b_ref[:, c * D:(c + 1) * D], 0.0)


def _finish(acc, b2d):
    blk = N // 10
    in_specs = (
        [pl.BlockSpec((blk, D), lambda i, c=c: (i + c * 10, 0))
         for c in range(C)]
        + [pl.BlockSpec((1, OUT), lambda i: (0, 0))]
    )
    return pl.pallas_call(
        _fin_body,
        grid=(10,),
        in_specs=in_specs,
        out_specs=pl.BlockSpec((blk, OUT), lambda i: (i, 0)),
        out_shape=jax.ShapeDtypeStruct((N, OUT), jnp.float32),
    )(acc, acc, acc, acc, b2d)


def kernel(node_features, edge_index, edge_attr, W, b):
    h = _project(node_features, W)
    src = edge_index[0].astype(jnp.int32)
    dst = edge_index[1].astype(jnp.int32)
    attr_flat = edge_attr.reshape(-1)
    acc = _sc_scatter(h, src, dst, attr_flat)
    return _finish(acc, b.reshape(1, OUT))


# speedup vs baseline: 1.8625x; 1.3753x over previous
import jax
import jax.numpy as jnp
from jax import lax
from jax.experimental import pallas as pl
from jax.experimental.pallas import tpu as pltpu
from jax.experimental.pallas import tpu_sc as plsc

N = 10000
E = 320000
D = 128
C = 4
OUT = D * C

NC = 2
NS = 16
L = 16

EDGES_PER_TILE = E // NS
BLK = 80
NBLK = EDGES_PER_TILE // BLK
NZCH = N // BLK
PASSES = C // NC


def _mm_body(x_ref, w_ref, o_ref):
    o_ref[...] = jnp.dot(x_ref[...], w_ref[...],
                         preferred_element_type=jnp.float32)


def _project(x, w):
    return pl.pallas_call(
        _mm_body,
        grid=(10,),
        in_specs=[
            pl.BlockSpec((N // 10, D), lambda i: (i, 0)),
            pl.BlockSpec((D, D), lambda i: (0, 0)),
        ],
        out_specs=pl.BlockSpec((N // 10, D), lambda i: (i, 0)),
        out_shape=jax.ShapeDtypeStruct((N, D), jnp.float32),
    )(x, w)


def _sc_scatter(h, src, dst, attr_flat):
    mesh = plsc.VectorSubcoreMesh(core_axis_name="c", subcore_axis_name="s",
                                  num_cores=NC, num_subcores=NS)

    def body(h_hbm, src_hbm, dst_hbm, attr_hbm, out_hbm,
             acc, srcb0, srcb1, dstb0, dstb1, dsts0, dsts1, attrb0, attrb1,
             rows0, rows1, msg0, msg1,
             isem0, isem1, gsem0, gsem1, ssem0, ssem1):
        core = lax.axis_index("c")
        s = lax.axis_index("s")
        zeros16 = jnp.zeros((L,), jnp.float32)
        srcb = [srcb0, srcb1]
        dstb = [dstb0, dstb1]
        dsts = [dsts0, dsts1]
        attrb = [attrb0, attrb1]
        rows = [rows0, rows1]
        msg = [msg0, msg1]
        isem = [isem0, isem1]
        gsem = [gsem0, gsem1]
        ssem = [ssem0, ssem1]

        def idx_base(b):
            return s * EDGES_PER_TILE + b * BLK

        def fire_idx(slot, b):
            base = idx_base(b)
            pltpu.async_copy(src_hbm.at[pl.ds(base, BLK)], srcb[slot],
                             isem[slot])
            pltpu.async_copy(dst_hbm.at[pl.ds(base, BLK)], dstb[slot],
                             isem[slot])
            pltpu.async_copy(attr_hbm.at[pl.ds(base * C, BLK * C)],
                             attrb[slot], isem[slot])

        def wait_idx(slot, b):
            base = idx_base(b)
            pltpu.make_async_copy(src_hbm.at[pl.ds(base, BLK)], srcb[slot],
                                  isem[slot]).wait()
            pltpu.make_async_copy(dst_hbm.at[pl.ds(base, BLK)], dstb[slot],
                                  isem[slot]).wait()
            pltpu.make_async_copy(attr_hbm.at[pl.ds(base * C, BLK * C)],
                                  attrb[slot], isem[slot]).wait()

        def fire_gather(slot):
            pltpu.async_copy(h_hbm.at[srcb[slot]], rows[slot], gsem[slot])

        def wait_gather(slot):
            pltpu.make_async_copy(h_hbm.at[srcb[slot]], rows[slot],
                                  gsem[slot]).wait()

        def fire_scatter(slot):
            pltpu.async_copy(msg[slot], acc.at[dsts[slot]], ssem[slot],
                             add=True)

        def wait_scatter(slot):
            pltpu.make_async_copy(msg[slot], acc.at[dsts[slot]],
                                  ssem[slot]).wait()

        for p in range(PASSES):
            ch = PASSES * core + p
            chv = jnp.full((L,), 1, dtype=jnp.int32) * ch

            def zrow(r, _):
                for j in range(D // L):
                    msg0[r, pl.ds(L * j, L)] = zeros16
                return 0
            lax.fori_loop(0, BLK, zrow, 0)

            def zcp(i, carry):
                k = s + NS * i

                @pl.when(k < NZCH)
                def _():
                    off = pl.multiple_of(k * BLK, 8)
                    pltpu.sync_copy(msg0, acc.at[pl.ds(off, BLK), :])
                return carry
            lax.fori_loop(0, (NZCH + NS - 1) // NS, zcp, 0)
            plsc.subcore_barrier()

            def compute_msg(slot):
                def edge_body(q, carry):
                    for u in range(4):
                        e = 4 * q + u
                        ev = jnp.full((L,), e, dtype=jnp.int32)
                        scale = plsc.load_gather(attrb[slot], [ev * C + chv])
                        for j in range(D // L):
                            msg[slot][e, pl.ds(L * j, L)] = (
                                rows[slot][e, pl.ds(L * j, L)] * scale)
                    return carry
                lax.fori_loop(0, BLK // 4, edge_body, 0)

            def snap_dst(slot):
                for g in range(BLK // L):
                    dsts[slot][pl.ds(L * g, L)] = dstb[slot][pl.ds(L * g, L)]

            fire_idx(0, 0)
            wait_idx(0, 0)
            fire_gather(0)
            fire_idx(1, 1)

            def pair_body(t, carry):
                for slot in range(2):
                    b = 2 * t + slot
                    other = 1 - slot
                    wait_gather(slot)

                    @pl.when(t > 0)
                    def _():
                        wait_scatter(slot)
                    compute_msg(slot)
                    snap_dst(slot)
                    fire_scatter(slot)

                    @pl.when(b + 2 < NBLK)
                    def _():
                        fire_idx(slot, b + 2)

                    @pl.when(b + 1 < NBLK)
                    def _():
                        wait_idx(other, b + 1)
                        fire_gather(other)
                return carry

            lax.fori_loop(0, NBLK // 2, pair_body, 0)
            wait_scatter(0)
            wait_scatter(1)
            plsc.subcore_barrier()

            def wcp(i, carry):
                k = s + NS * i

                @pl.when(k < NZCH)
                def _():
                    off = pl.multiple_of(k * BLK, 8)
                    dof = pl.multiple_of(ch * N + k * BLK, 8)
                    pltpu.sync_copy(acc.at[pl.ds(off, BLK), :],
                                    out_hbm.at[pl.ds(dof, BLK), :])
                return carry
            lax.fori_loop(0, (NZCH + NS - 1) // NS, wcp, 0)
            if p + 1 < PASSES:
                plsc.subcore_barrier()

    f = pl.kernel(
        body,
        out_type=jax.ShapeDtypeStruct((C * N, D), jnp.float32),
        mesh=mesh,
        compiler_params=pltpu.CompilerParams(needs_layout_passes=False),
        scratch_types=dict(
            acc=pltpu.VMEM_SHARED((N, D), jnp.float32),
            srcb0=pltpu.VMEM((BLK,), jnp.int32),
            srcb1=pltpu.VMEM((BLK,), jnp.int32),
            dstb0=pltpu.VMEM((BLK,), jnp.int32),
            dstb1=pltpu.VMEM((BLK,), jnp.int32),
            dsts0=pltpu.VMEM((BLK,), jnp.int32),
            dsts1=pltpu.VMEM((BLK,), jnp.int32),
            attrb0=pltpu.VMEM((BLK * C,), jnp.float32),
            attrb1=pltpu.VMEM((BLK * C,), jnp.float32),
            rows0=pltpu.VMEM((BLK, D), jnp.float32),
            rows1=pltpu.VMEM((BLK, D), jnp.float32),
            msg0=pltpu.VMEM((BLK, D), jnp.float32),
            msg1=pltpu.VMEM((BLK, D), jnp.float32),
            isem0=pltpu.SemaphoreType.DMA,
            isem1=pltpu.SemaphoreType.DMA,
            gsem0=pltpu.SemaphoreType.DMA,
            gsem1=pltpu.SemaphoreType.DMA,
            ssem0=pltpu.SemaphoreType.DMA,
            ssem1=pltpu.SemaphoreType.DMA,
        ),
    )
    return f(h, src, dst, attr_flat)


def _fin_body(a0, a1, a2, a3, b_ref, o_ref):
    av = [a0, a1, a2, a3]
    for c in range(C):
        o_ref[:, c * D:(c + 1) * D] = jnp.maximum(
            av[c][...] + b_ref[:, c * D:(c + 1) * D], 0.0)


def _finish(acc, b2d):
    blk = N // 10
    in_specs = (
        [pl.BlockSpec((blk, D), lambda i, c=c: (i + c * 10, 0))
         for c in range(C)]
        + [pl.BlockSpec((1, OUT), lambda i: (0, 0))]
    )
    return pl.pallas_call(
        _fin_body,
        grid=(10,),
        in_specs=in_specs,
        out_specs=pl.BlockSpec((blk, OUT), lambda i: (i, 0)),
        out_shape=jax.ShapeDtypeStruct((N, OUT), jnp.float32),
    )(acc, acc, acc, acc, b2d)


def kernel(node_features, edge_index, edge_attr, W, b):
    h = _project(node_features, W)
    src = edge_index[0].astype(jnp.int32)
    dst = edge_index[1].astype(jnp.int32)
    attr_flat = edge_attr.reshape(-1)
    acc = _sc_scatter(h, src, dst, attr_flat)
    return _finish(acc, b.reshape(1, OUT))


# speedup vs baseline: 3.5741x; 1.9189x over previous
import jax
import jax.numpy as jnp
from jax import lax
from jax.experimental import pallas as pl
from jax.experimental.pallas import tpu as pltpu
from jax.experimental.pallas import tpu_sc as plsc

N = 10000
E = 320000
D = 128
C = 4
OUT = D * C

NC = 2
NS = 16
L = 16

EDGES_PER_TILE = E // NS
BLK = 80
NBLK = EDGES_PER_TILE // BLK
NZCH = N // BLK
PASSES = C // NC


def _mm_body(x_ref, w_ref, o_ref):
    o_ref[...] = jnp.dot(x_ref[...], w_ref[...],
                         preferred_element_type=jnp.float32)


def _project(x, w):
    return pl.pallas_call(
        _mm_body,
        grid=(10,),
        in_specs=[
            pl.BlockSpec((N // 10, D), lambda i: (i, 0)),
            pl.BlockSpec((D, D), lambda i: (0, 0)),
        ],
        out_specs=pl.BlockSpec((N // 10, D), lambda i: (i, 0)),
        out_shape=jax.ShapeDtypeStruct((N, D), jnp.float32),
    )(x, w)


def _sc_scatter(h, src, dst, attr_flat):
    mesh = plsc.VectorSubcoreMesh(core_axis_name="c", subcore_axis_name="s",
                                  num_cores=NC, num_subcores=NS)

    def body(h_hbm, src_hbm, dst_hbm, attr_hbm, out_hbm,
             acc, srcb0, srcb1, srcb2, dstb0, dstb1, dstb2,
             dsts0, dsts1, dsts2, attrb0, attrb1, attrb2,
             rows0, rows1, rows2,
             isem0, isem1, isem2, gsem0, gsem1, gsem2,
             ssem0, ssem1, ssem2):
        core = lax.axis_index("c")
        s = lax.axis_index("s")
        zeros16 = jnp.zeros((L,), jnp.float32)
        srcb = [srcb0, srcb1, srcb2]
        dstb = [dstb0, dstb1, dstb2]
        dsts = [dsts0, dsts1, dsts2]
        attrb = [attrb0, attrb1, attrb2]
        rows = [rows0, rows1, rows2]
        isem = [isem0, isem1, isem2]
        gsem = [gsem0, gsem1, gsem2]
        ssem = [ssem0, ssem1, ssem2]
        Q = 3

        def idx_base(b):
            return s * EDGES_PER_TILE + b * BLK

        def fire_idx(slot, b):
            base = idx_base(b)
            pltpu.async_copy(src_hbm.at[pl.ds(base, BLK)], srcb[slot],
                             isem[slot])
            pltpu.async_copy(dst_hbm.at[pl.ds(base, BLK)], dstb[slot],
                             isem[slot])
            pltpu.async_copy(attr_hbm.at[pl.ds(base * C, BLK * C)],
                             attrb[slot], isem[slot])

        def wait_idx(slot, b):
            base = idx_base(b)
            pltpu.make_async_copy(src_hbm.at[pl.ds(base, BLK)], srcb[slot],
                                  isem[slot]).wait()
            pltpu.make_async_copy(dst_hbm.at[pl.ds(base, BLK)], dstb[slot],
                                  isem[slot]).wait()
            pltpu.make_async_copy(attr_hbm.at[pl.ds(base * C, BLK * C)],
                                  attrb[slot], isem[slot]).wait()

        def fire_gather(slot):
            pltpu.async_copy(h_hbm.at[srcb[slot]], rows[slot], gsem[slot])

        def wait_gather(slot):
            pltpu.make_async_copy(h_hbm.at[srcb[slot]], rows[slot],
                                  gsem[slot]).wait()

        def fire_scatter(slot):
            pltpu.async_copy(rows[slot], acc.at[dsts[slot]], ssem[slot],
                             add=True)

        def wait_scatter(slot):
            pltpu.make_async_copy(rows[slot], acc.at[dsts[slot]],
                                  ssem[slot]).wait()

        for p in range(PASSES):
            ch = PASSES * core + p
            chv = jnp.full((L,), 1, dtype=jnp.int32) * ch

            def zrow(r, _):
                for j in range(D // L):
                    rows0[r, pl.ds(L * j, L)] = zeros16
                return 0
            lax.fori_loop(0, BLK, zrow, 0)

            def zcp(i, carry):
                k = s + NS * i

                @pl.when(k < NZCH)
                def _():
                    off = pl.multiple_of(k * BLK, 8)
                    pltpu.sync_copy(rows0, acc.at[pl.ds(off, BLK), :])
                return carry
            lax.fori_loop(0, (NZCH + NS - 1) // NS, zcp, 0)
            plsc.subcore_barrier()

            def compute_scale(slot):
                def edge_body(q, carry):
                    for u in range(4):
                        e = 4 * q + u
                        ev = jnp.full((L,), e, dtype=jnp.int32)
                        scale = plsc.load_gather(attrb[slot], [ev * C + chv])
                        for j in range(D // L):
                            rows[slot][e, pl.ds(L * j, L)] = (
                                rows[slot][e, pl.ds(L * j, L)] * scale)
                    return carry
                lax.fori_loop(0, BLK // 4, edge_body, 0)

            def snap_dst(slot):
                for g in range(BLK // L):
                    dsts[slot][pl.ds(L * g, L)] = dstb[slot][pl.ds(L * g, L)]

            def halfstep(t, u):
                b = Q * t + u
                nxt = (u + 1) % Q
                wait_gather(u)
                compute_scale(u)
                snap_dst(u)
                fire_scatter(u)

                @pl.when(b + Q < NBLK)
                def _():
                    fire_idx(u, b + Q)

                @pl.when(b + 1 < NBLK)
                def _():
                    @pl.when(b >= 2)
                    def _():
                        wait_scatter(nxt)
                    wait_idx(nxt, b + 1)
                    fire_gather(nxt)

            fire_idx(0, 0)
            fire_idx(1, 1)
            fire_idx(2, 2)
            wait_idx(0, 0)
            fire_gather(0)

            def trip_body(t, carry):
                for u in range(Q):
                    halfstep(t, u)
                return carry

            lax.fori_loop(0, NBLK // Q, trip_body, 0)
            halfstep(NBLK // Q, 0)
            wait_scatter(1)
            wait_scatter(2)
            wait_scatter(0)
            plsc.subcore_barrier()

            def wcp(i, carry):
                k = s + NS * i

                @pl.when(k < NZCH)
                def _():
                    off = pl.multiple_of(k * BLK, 8)
                    dof = pl.multiple_of(ch * N + k * BLK, 8)
                    pltpu.sync_copy(acc.at[pl.ds(off, BLK), :],
                                    out_hbm.at[pl.ds(dof, BLK), :])
                return carry
            lax.fori_loop(0, (NZCH + NS - 1) // NS, wcp, 0)
            if p + 1 < PASSES:
                plsc.subcore_barrier()

    f = pl.kernel(
        body,
        out_type=jax.ShapeDtypeStruct((C * N, D), jnp.float32),
        mesh=mesh,
        compiler_params=pltpu.CompilerParams(needs_layout_passes=False),
        scratch_types=dict(
            acc=pltpu.VMEM_SHARED((N, D), jnp.float32),
            srcb0=pltpu.VMEM((BLK,), jnp.int32),
            srcb1=pltpu.VMEM((BLK,), jnp.int32),
            srcb2=pltpu.VMEM((BLK,), jnp.int32),
            dstb0=pltpu.VMEM((BLK,), jnp.int32),
            dstb1=pltpu.VMEM((BLK,), jnp.int32),
            dstb2=pltpu.VMEM((BLK,), jnp.int32),
            dsts0=pltpu.VMEM((BLK,), jnp.int32),
            dsts1=pltpu.VMEM((BLK,), jnp.int32),
            dsts2=pltpu.VMEM((BLK,), jnp.int32),
            attrb0=pltpu.VMEM((BLK * C,), jnp.float32),
            attrb1=pltpu.VMEM((BLK * C,), jnp.float32),
            attrb2=pltpu.VMEM((BLK * C,), jnp.float32),
            rows0=pltpu.VMEM((BLK, D), jnp.float32),
            rows1=pltpu.VMEM((BLK, D), jnp.float32),
            rows2=pltpu.VMEM((BLK, D), jnp.float32),
            isem0=pltpu.SemaphoreType.DMA,
            isem1=pltpu.SemaphoreType.DMA,
            isem2=pltpu.SemaphoreType.DMA,
            gsem0=pltpu.SemaphoreType.DMA,
            gsem1=pltpu.SemaphoreType.DMA,
            gsem2=pltpu.SemaphoreType.DMA,
            ssem0=pltpu.SemaphoreType.DMA,
            ssem1=pltpu.SemaphoreType.DMA,
            ssem2=pltpu.SemaphoreType.DMA,
        ),
    )
    return f(h, src, dst, attr_flat)


def _fin_body(a0, a1, a2, a3, b_ref, o_ref):
    av = [a0, a1, a2, a3]
    for c in range(C):
        o_ref[:, c * D:(c + 1) * D] = jnp.maximum(
            av[c][...] + b_ref[:, c * D:(c + 1) * D], 0.0)


def _finish(acc, b2d):
    blk = N // 10
    in_specs = (
        [pl.BlockSpec((blk, D), lambda i, c=c: (i + c * 10, 0))
         for c in range(C)]
        + [pl.BlockSpec((1, OUT), lambda i: (0, 0))]
    )
    return pl.pallas_call(
        _fin_body,
        grid=(10,),
        in_specs=in_specs,
        out_specs=pl.BlockSpec((blk, OUT), lambda i: (i, 0)),
        out_shape=jax.ShapeDtypeStruct((N, OUT), jnp.float32),
    )(acc, acc, acc, acc, b2d)


def kernel(node_features, edge_index, edge_attr, W, b):
    h = _project(node_features, W)
    src = edge_index[0].astype(jnp.int32)
    dst = edge_index[1].astype(jnp.int32)
    attr_flat = edge_attr.reshape(-1)
    acc = _sc_scatter(h, src, dst, attr_flat)
    return _finish(acc, b.reshape(1, OUT))


# speedup vs baseline: 4.0536x; 1.1342x over previous
import jax
import jax.numpy as jnp
from jax import lax
from jax.experimental import pallas as pl
from jax.experimental.pallas import tpu as pltpu
from jax.experimental.pallas import tpu_sc as plsc

N = 10000
E = 320000
D = 128
C = 4
OUT = D * C

NC = 2
NS = 16
L = 16

EDGES_PER_TILE = E // NS
BLK = 80
NBLK = EDGES_PER_TILE // BLK
NZCH = N // BLK
PASSES = C // NC


def _mm_body(x_ref, w_ref, o_ref):
    o_ref[...] = jnp.dot(x_ref[...], w_ref[...],
                         preferred_element_type=jnp.float32)


def _project(x, w):
    return pl.pallas_call(
        _mm_body,
        grid=(10,),
        in_specs=[
            pl.BlockSpec((N // 10, D), lambda i: (i, 0)),
            pl.BlockSpec((D, D), lambda i: (0, 0)),
        ],
        out_specs=pl.BlockSpec((N // 10, D), lambda i: (i, 0)),
        out_shape=jax.ShapeDtypeStruct((N, D), jnp.float32),
    )(x, w)


def _sc_scatter(h, src, dst, attr_flat):
    mesh = plsc.VectorSubcoreMesh(core_axis_name="c", subcore_axis_name="s",
                                  num_cores=NC, num_subcores=NS)

    def body(h_hbm, src_hbm, dst_hbm, attr_hbm, out_hbm,
             acc, srcb0, srcb1, srcb2, dstb0, dstb1, dstb2,
             dsts0, dsts1, dsts2, attrb0, attrb1, attrb2,
             rows0, rows1, rows2,
             isem0, isem1, isem2, gsem0, gsem1, gsem2,
             ssem0, ssem1, ssem2):
        core = lax.axis_index("c")
        s = lax.axis_index("s")
        zeros16 = jnp.zeros((L,), jnp.float32)
        srcb = [srcb0, srcb1, srcb2]
        dstb = [dstb0, dstb1, dstb2]
        dsts = [dsts0, dsts1, dsts2]
        attrb = [attrb0, attrb1, attrb2]
        rows = [rows0, rows1, rows2]
        isem = [isem0, isem1, isem2]
        gsem = [gsem0, gsem1, gsem2]
        ssem = [ssem0, ssem1, ssem2]
        Q = 3

        def idx_base(b):
            return s * EDGES_PER_TILE + b * BLK

        def fire_idx(slot, b):
            base = idx_base(b)
            pltpu.async_copy(src_hbm.at[pl.ds(base, BLK)], srcb[slot],
                             isem[slot])
            pltpu.async_copy(dst_hbm.at[pl.ds(base, BLK)], dstb[slot],
                             isem[slot])
            pltpu.async_copy(attr_hbm.at[pl.ds(base * C, BLK * C)],
                             attrb[slot], isem[slot])

        def wait_idx(slot, b):
            base = idx_base(b)
            pltpu.make_async_copy(src_hbm.at[pl.ds(base, BLK)], srcb[slot],
                                  isem[slot]).wait()
            pltpu.make_async_copy(dst_hbm.at[pl.ds(base, BLK)], dstb[slot],
                                  isem[slot]).wait()
            pltpu.make_async_copy(attr_hbm.at[pl.ds(base * C, BLK * C)],
                                  attrb[slot], isem[slot]).wait()

        def fire_gather(slot):
            pltpu.async_copy(h_hbm.at[srcb[slot]], rows[slot], gsem[slot])

        def wait_gather(slot):
            pltpu.make_async_copy(h_hbm.at[srcb[slot]], rows[slot],
                                  gsem[slot]).wait()

        def fire_scatter(slot):
            pltpu.async_copy(rows[slot], acc.at[dsts[slot]], ssem[slot],
                             add=True)

        def wait_scatter(slot):
            pltpu.make_async_copy(rows[slot], acc.at[dsts[slot]],
                                  ssem[slot]).wait()

        for p in range(PASSES):
            ch = PASSES * core + p
            chv = jnp.full((L,), 1, dtype=jnp.int32) * ch

            def zrow(r, _):
                for j in range(D // L):
                    rows0[r, pl.ds(L * j, L)] = zeros16
                return 0
            lax.fori_loop(0, BLK, zrow, 0)

            def zcp(i, carry):
                k = s + NS * i

                @pl.when(k < NZCH)
                def _():
                    off = pl.multiple_of(k * BLK, 8)
                    pltpu.sync_copy(rows0, acc.at[pl.ds(off, BLK), :])
                return carry
            lax.fori_loop(0, (NZCH + NS - 1) // NS, zcp, 0)
            plsc.subcore_barrier()

            def compute_scale(slot):
                def edge_body(q, carry):
                    scales = []
                    for u in range(8):
                        e = 8 * q + u
                        ev = jnp.full((L,), e * C, dtype=jnp.int32)
                        scales.append(
                            plsc.load_gather(attrb[slot], [ev + chv]))
                    for u in range(8):
                        e = 8 * q + u
                        for j in range(D // L):
                            rows[slot][e, pl.ds(L * j, L)] = (
                                rows[slot][e, pl.ds(L * j, L)] * scales[u])
                    return carry
                lax.fori_loop(0, BLK // 8, edge_body, 0)

            def snap_dst(slot):
                for g in range(BLK // L):
                    dsts[slot][pl.ds(L * g, L)] = dstb[slot][pl.ds(L * g, L)]

            def halfstep(t, u):
                b = Q * t + u
                nxt = (u + 1) % Q
                wait_gather(u)
                compute_scale(u)
                snap_dst(u)
                fire_scatter(u)

                @pl.when(b + Q < NBLK)
                def _():
                    fire_idx(u, b + Q)

                @pl.when(b + 1 < NBLK)
                def _():
                    @pl.when(b >= 2)
                    def _():
                        wait_scatter(nxt)
                    wait_idx(nxt, b + 1)
                    fire_gather(nxt)

            fire_idx(0, 0)
            fire_idx(1, 1)
            fire_idx(2, 2)
            wait_idx(0, 0)
            fire_gather(0)

            def trip_body(t, carry):
                for u in range(Q):
                    halfstep(t, u)
                return carry

            lax.fori_loop(0, NBLK // Q, trip_body, 0)
            halfstep(NBLK // Q, 0)
            wait_scatter(1)
            wait_scatter(2)
            wait_scatter(0)
            plsc.subcore_barrier()

            def wcp(i, carry):
                k = s + NS * i

                @pl.when(k < NZCH)
                def _():
                    off = pl.multiple_of(k * BLK, 8)
                    dof = pl.multiple_of(ch * N + k * BLK, 8)
                    pltpu.sync_copy(acc.at[pl.ds(off, BLK), :],
                                    out_hbm.at[pl.ds(dof, BLK), :])
                return carry
            lax.fori_loop(0, (NZCH + NS - 1) // NS, wcp, 0)
            if p + 1 < PASSES:
                plsc.subcore_barrier()

    f = pl.kernel(
        body,
        out_type=jax.ShapeDtypeStruct((C * N, D), jnp.float32),
        mesh=mesh,
        compiler_params=pltpu.CompilerParams(needs_layout_passes=False),
        scratch_types=dict(
            acc=pltpu.VMEM_SHARED((N, D), jnp.float32),
            srcb0=pltpu.VMEM((BLK,), jnp.int32),
            srcb1=pltpu.VMEM((BLK,), jnp.int32),
            srcb2=pltpu.VMEM((BLK,), jnp.int32),
            dstb0=pltpu.VMEM((BLK,), jnp.int32),
            dstb1=pltpu.VMEM((BLK,), jnp.int32),
            dstb2=pltpu.VMEM((BLK,), jnp.int32),
            dsts0=pltpu.VMEM((BLK,), jnp.int32),
            dsts1=pltpu.VMEM((BLK,), jnp.int32),
            dsts2=pltpu.VMEM((BLK,), jnp.int32),
            attrb0=pltpu.VMEM((BLK * C,), jnp.float32),
            attrb1=pltpu.VMEM((BLK * C,), jnp.float32),
            attrb2=pltpu.VMEM((BLK * C,), jnp.float32),
            rows0=pltpu.VMEM((BLK, D), jnp.float32),
            rows1=pltpu.VMEM((BLK, D), jnp.float32),
            rows2=pltpu.VMEM((BLK, D), jnp.float32),
            isem0=pltpu.SemaphoreType.DMA,
            isem1=pltpu.SemaphoreType.DMA,
            isem2=pltpu.SemaphoreType.DMA,
            gsem0=pltpu.SemaphoreType.DMA,
            gsem1=pltpu.SemaphoreType.DMA,
            gsem2=pltpu.SemaphoreType.DMA,
            ssem0=pltpu.SemaphoreType.DMA,
            ssem1=pltpu.SemaphoreType.DMA,
            ssem2=pltpu.SemaphoreType.DMA,
        ),
    )
    return f(h, src, dst, attr_flat)


def _fin_body(a0, a1, a2, a3, b_ref, o_ref):
    av = [a0, a1, a2, a3]
    for c in range(C):
        o_ref[:, c * D:(c + 1) * D] = jnp.maximum(
            av[c][...] + b_ref[:, c * D:(c + 1) * D], 0.0)


def _finish(acc, b2d):
    blk = N // 10
    in_specs = (
        [pl.BlockSpec((blk, D), lambda i, c=c: (i + c * 10, 0))
         for c in range(C)]
        + [pl.BlockSpec((1, OUT), lambda i: (0, 0))]
    )
    return pl.pallas_call(
        _fin_body,
        grid=(10,),
        in_specs=in_specs,
        out_specs=pl.BlockSpec((blk, OUT), lambda i: (i, 0)),
        out_shape=jax.ShapeDtypeStruct((N, OUT), jnp.float32),
    )(acc, acc, acc, acc, b2d)


def kernel(node_features, edge_index, edge_attr, W, b):
    h = _project(node_features, W)
    src = edge_index[0].astype(jnp.int32)
    dst = edge_index[1].astype(jnp.int32)
    attr_flat = edge_attr.reshape(-1)
    acc = _sc_scatter(h, src, dst, attr_flat)
    return _finish(acc, b.reshape(1, OUT))
